# Initial kernel scaffold; baseline (speedup 1.0000x reference)
import jax
import jax.numpy as jnp
from jax.experimental import pallas as pl


def kernel(nf, edge_index, node_type, W1, gamma1, beta1, W2, gamma2, beta2):
    raise NotImplementedError("write your pallas kernel here")


# speedup vs baseline: 3.7865x; 3.7865x over previous
import functools

import jax
import jax.numpy as jnp
from jax import lax
from jax.experimental import pallas as pl
from jax.experimental.pallas import tpu as pltpu
from jax.experimental.pallas import tpu_sc as plsc

N_TASK = 5000
N_AG = 5000
D = 128
DEG = 64
E = N_AG * DEG
FIN_TASK_TYPE = 3
EPS = 1e-5
NEG_SLOPE = 0.01

NC, NS = 2, 16
NW = NC * NS
A_PER = 160
N_AG_PAD = NW * A_PER
E_PAD = N_AG_PAD * DEG
C = D // 16

_SC_MESH = plsc.VectorSubcoreMesh(
    core_axis_name="c", subcore_axis_name="s", num_cores=NC, num_subcores=NS)


def _proj_body(nt_ref, na_ref, ws_ref, wd_ref, t_ref, a_ref):
    t_ref[...] = jnp.dot(nt_ref[...], ws_ref[...],
                         preferred_element_type=jnp.float32)
    a_ref[...] = jnp.dot(na_ref[...], wd_ref[...],
                         preferred_element_type=jnp.float32)


def _proj(nf_t, nf_a, w1s, w1d):
    blk = 1000
    return pl.pallas_call(
        _proj_body,
        grid=(N_TASK // blk,),
        in_specs=[
            pl.BlockSpec((blk, D), lambda i: (i, 0)),
            pl.BlockSpec((blk, D), lambda i: (i, 0)),
            pl.BlockSpec((D, D), lambda i: (0, 0)),
            pl.BlockSpec((D, D), lambda i: (0, 0)),
        ],
        out_specs=[
            pl.BlockSpec((blk, D), lambda i: (i, 0)),
            pl.BlockSpec((blk, D), lambda i: (i, 0)),
        ],
        out_shape=[
            jax.ShapeDtypeStruct((N_TASK, D), jnp.float32),
            jax.ShapeDtypeStruct((N_AG, D), jnp.float32),
        ],
    )(nf_t, nf_a, w1s, w1d)


def _k1_body(tproj, aproj, srcpad, part, idx_v, ap_v, rows_v, acc_v):
    wid = lax.axis_index("s") * NC + lax.axis_index("c")
    base_a = wid * A_PER
    pltpu.sync_copy(srcpad.at[pl.ds(base_a * DEG, A_PER * DEG)], idx_v)
    pltpu.sync_copy(aproj.at[pl.ds(base_a, A_PER)], ap_v)

    zero = jnp.zeros((16,), jnp.float32)
    for c in range(2 * C):
        acc_v[pl.ds(c * 16, 16)] = zero

    def agent_body(a, carry):
        pltpu.sync_copy(tproj.at[idx_v.at[pl.ds(a * DEG, DEG)]], rows_v)
        valid = (base_a + a) < N_AG

        @pl.when(valid)
        def _():
            arow = [ap_v[a, pl.ds(c * 16, 16)] for c in range(C)]

            def edge_body(j, sq):
                s, q = sq
                s2, q2 = [], []
                for c in range(C):
                    r = rows_v[j, pl.ds(c * 16, 16)]
                    h = r + arow[c]
                    s2.append(s[c] + h)
                    q2.append(q[c] + h * h)
                return (tuple(s2), tuple(q2))

            init = (
                tuple(acc_v[pl.ds(c * 16, 16)] for c in range(C)),
                tuple(acc_v[pl.ds((C + c) * 16, 16)] for c in range(C)),
            )
            fs, fq = lax.fori_loop(0, DEG, edge_body, init)
            for c in range(C):
                acc_v[pl.ds(c * 16, 16)] = fs[c]
                acc_v[pl.ds((C + c) * 16, 16)] = fq[c]

        return carry

    lax.fori_loop(0, A_PER, agent_body, jnp.int32(0))
    pltpu.sync_copy(acc_v, part.at[wid])


@functools.partial(
    pl.kernel,
    out_type=jax.ShapeDtypeStruct((NW, 2 * D), jnp.float32),
    mesh=_SC_MESH,
    compiler_params=pltpu.CompilerParams(needs_layout_passes=False),
    scratch_types=[
        pltpu.VMEM((A_PER * DEG,), jnp.int32),
        pltpu.VMEM((A_PER, D), jnp.float32),
        pltpu.VMEM((DEG, D), jnp.float32),
        pltpu.VMEM((2 * D,), jnp.float32),
    ],
)
def _k1(tproj, aproj, srcpad, part, idx_v, ap_v, rows_v, acc_v):
    _k1_body(tproj, aproj, srcpad, part, idx_v, ap_v, rows_v, acc_v)


def _k2_body(tproj, aproj, srcpad, kbw, fin, s_out, f_out, part2,
             idx_v, ap_v, rows_v, accb_v, sbuf_v, fbuf_v, kbw_v, fin_v,
             acc2_v):
    wid = lax.axis_index("s") * NC + lax.axis_index("c")
    base_a = wid * A_PER
    pltpu.sync_copy(srcpad.at[pl.ds(base_a * DEG, A_PER * DEG)], idx_v)
    pltpu.sync_copy(aproj.at[pl.ds(base_a, A_PER)], ap_v)
    pltpu.sync_copy(kbw, kbw_v)
    pltpu.sync_copy(fin, fin_v)

    zero = jnp.zeros((16,), jnp.float32)
    acc2_v[pl.ds(0, 16)] = zero
    acc2_v[pl.ds(16, 16)] = zero

    k1v = [kbw_v[pl.ds(c * 16, 16)] for c in range(C)]
    b1v = [kbw_v[pl.ds((C + c) * 16, 16)] for c in range(C)]
    w2v = [kbw_v[pl.ds((2 * C + c) * 16, 16)] for c in range(C)]
    lanes = lax.iota(jnp.int32, 16)

    def agent_body(a, carry):
        pltpu.sync_copy(tproj.at[idx_v.at[pl.ds(a * DEG, DEG)]], rows_v)
        valid = (base_a + a) < N_AG

        @pl.when(valid)
        def _():
            cav = [ap_v[a, pl.ds(c * 16, 16)] * k1v[c] + b1v[c]
                   for c in range(C)]

            def edge_body(j, cy):
                acc0 = zero
                acc1 = zero
                for c in range(C):
                    r = rows_v[j, pl.ds(c * 16, 16)]
                    hn = r * k1v[c] + cav[c]
                    lr = jnp.maximum(hn, hn * NEG_SLOPE)
                    t = lr * w2v[c]
                    if c % 2 == 0:
                        acc0 = acc0 + t
                    else:
                        acc1 = acc1 + t
                accb_v[pl.ds(j * 16, 16)] = acc0 + acc1
                return cy

            lax.fori_loop(0, DEG, edge_body, jnp.int32(0))

            ss = acc2_v[pl.ds(0, 16)]
            ss2 = acc2_v[pl.ds(16, 16)]
            for g in range(DEG // 16):
                baseidx = (g * 16) * 16 + lanes * 16
                sa = plsc.load_gather(accb_v, [baseidx])
                sb = plsc.load_gather(accb_v, [baseidx + 1])
                for m in range(2, 16, 2):
                    sa = sa + plsc.load_gather(accb_v, [baseidx + m])
                    sb = sb + plsc.load_gather(accb_v, [baseidx + m + 1])
                s_grp = sa + sb
                sbuf_v[pl.ds(g * 16, 16)] = s_grp
                srcv = idx_v[pl.ds(a * DEG + g * 16, 16)]
                fbuf_v[pl.ds(g * 16, 16)] = plsc.load_gather(fin_v, [srcv])
                ss = ss + s_grp
                ss2 = ss2 + s_grp * s_grp
            acc2_v[pl.ds(0, 16)] = ss
            acc2_v[pl.ds(16, 16)] = ss2

            pltpu.sync_copy(sbuf_v, s_out.at[pl.ds((base_a + a) * DEG, DEG)])
            pltpu.sync_copy(fbuf_v, f_out.at[pl.ds((base_a + a) * DEG, DEG)])

        return carry

    lax.fori_loop(0, A_PER, agent_body, jnp.int32(0))
    pltpu.sync_copy(acc2_v, part2.at[wid])


@functools.partial(
    pl.kernel,
    out_type=(
        jax.ShapeDtypeStruct((E_PAD,), jnp.float32),
        jax.ShapeDtypeStruct((E_PAD,), jnp.float32),
        jax.ShapeDtypeStruct((NW, 32), jnp.float32),
    ),
    mesh=_SC_MESH,
    compiler_params=pltpu.CompilerParams(needs_layout_passes=False),
    scratch_types=[
        pltpu.VMEM((A_PER * DEG,), jnp.int32),
        pltpu.VMEM((A_PER, D), jnp.float32),
        pltpu.VMEM((DEG, D), jnp.float32),
        pltpu.VMEM((DEG * 16,), jnp.float32),
        pltpu.VMEM((DEG,), jnp.float32),
        pltpu.VMEM((DEG,), jnp.float32),
        pltpu.VMEM((3 * D,), jnp.float32),
        pltpu.VMEM((N_TASK,), jnp.float32),
        pltpu.VMEM((32,), jnp.float32),
    ],
)
def _k2(tproj, aproj, srcpad, kbw, fin, s_out, f_out, part2,
        idx_v, ap_v, rows_v, accb_v, sbuf_v, fbuf_v, kbw_v, fin_v, acc2_v):
    _k2_body(tproj, aproj, srcpad, kbw, fin, s_out, f_out, part2,
             idx_v, ap_v, rows_v, accb_v, sbuf_v, fbuf_v, kbw_v, fin_v,
             acc2_v)


def _fin_body(s_ref, f_ref, scal_ref, o_ref):
    k2 = scal_ref[0]
    b2 = scal_ref[1]
    o_ref[...] = jnp.where(f_ref[...] > 0.5, -jnp.inf,
                           s_ref[...] * k2 + b2)


def _final(s2, f2, scal):
    rows = E // D
    return pl.pallas_call(
        _fin_body,
        grid=(1,),
        in_specs=[
            pl.BlockSpec((rows, D), lambda i: (0, 0)),
            pl.BlockSpec((rows, D), lambda i: (0, 0)),
            pl.BlockSpec(memory_space=pltpu.SMEM),
        ],
        out_specs=pl.BlockSpec((rows, D), lambda i: (0, 0)),
        out_shape=jax.ShapeDtypeStruct((rows, D), jnp.float32),
    )(s2, f2, scal)


def kernel(nf, edge_index, node_type, W1, gamma1, beta1, W2, gamma2, beta2):
    src = edge_index[0].astype(jnp.int32)
    nf_t = nf[:N_TASK]
    nf_a = nf[N_TASK:]
    w1s = W1[:D]
    w1d = W1[D:]

    t_proj, a_proj = _proj(nf_t, nf_a, w1s, w1d)

    src_pad = jnp.pad(src, (0, E_PAD - E))
    a_proj_pad = jnp.pad(a_proj, ((0, N_AG_PAD - N_AG), (0, 0)))

    part = _k1(t_proj, a_proj_pad, src_pad)
    sums = part[:, :D].sum(axis=0)
    sqs = part[:, D:].sum(axis=0)
    mu1 = sums / E
    var1 = sqs / E - mu1 * mu1
    k1 = gamma1 / jnp.sqrt(var1 + EPS)
    b1 = beta1 - mu1 * k1

    kbw = jnp.concatenate([k1, b1, W2[:, 0]])
    fin = (node_type[:N_TASK] == FIN_TASK_TYPE).astype(jnp.float32)

    s_pad, f_pad, part2 = _k2(t_proj, a_proj_pad, src_pad, kbw, fin)
    ss = part2[:, :16].sum()
    ss2 = part2[:, 16:].sum()
    mu2 = ss / E
    var2 = ss2 / E - mu2 * mu2
    k2 = gamma2[0] / jnp.sqrt(var2 + EPS)
    b2 = beta2[0] - mu2 * k2
    scal = jnp.stack([k2, b2])

    s2 = s_pad[:E].reshape(E // D, D)
    f2 = f_pad[:E].reshape(E // D, D)
    out = _final(s2, f2, scal)
    return out.reshape(N_AG, DEG)


# speedup vs baseline: 4.3914x; 1.1598x over previous
import functools

import jax
import jax.numpy as jnp
from jax import lax
from jax.experimental import pallas as pl
from jax.experimental.pallas import tpu as pltpu
from jax.experimental.pallas import tpu_sc as plsc

N_TASK = 5000
N_AG = 5000
D = 128
DEG = 64
E = N_AG * DEG
FIN_TASK_TYPE = 3
EPS = 1e-5
NEG_SLOPE = 0.01

NC, NS = 2, 16
NW = NC * NS
A_PER = 160
N_AG_PAD = NW * A_PER
E_PAD = N_AG_PAD * DEG
C = D // 16

_SC_MESH = plsc.VectorSubcoreMesh(
    core_axis_name="c", subcore_axis_name="s", num_cores=NC, num_subcores=NS)


def _proj_body(nt_ref, na_ref, ws_ref, wd_ref, t_ref, a_ref):
    t_ref[...] = jnp.dot(nt_ref[...], ws_ref[...],
                         preferred_element_type=jnp.float32)
    a_ref[...] = jnp.dot(na_ref[...], wd_ref[...],
                         preferred_element_type=jnp.float32)


def _proj(nf_t, nf_a, w1s, w1d):
    blk = 1000
    return pl.pallas_call(
        _proj_body,
        grid=(N_TASK // blk,),
        in_specs=[
            pl.BlockSpec((blk, D), lambda i: (i, 0)),
            pl.BlockSpec((blk, D), lambda i: (i, 0)),
            pl.BlockSpec((D, D), lambda i: (0, 0)),
            pl.BlockSpec((D, D), lambda i: (0, 0)),
        ],
        out_specs=[
            pl.BlockSpec((blk, D), lambda i: (i, 0)),
            pl.BlockSpec((blk, D), lambda i: (i, 0)),
        ],
        out_shape=[
            jax.ShapeDtypeStruct((N_TASK, D), jnp.float32),
            jax.ShapeDtypeStruct((N_AG, D), jnp.float32),
        ],
    )(nf_t, nf_a, w1s, w1d)


AB = 2
NB = A_PER // AB
BROWS = AB * DEG


def _k1_body(tproj, aproj, srcpad, part, idx_v, ap_v, rows0_v, rows1_v, acc_v,
             sem0, sem1):
    wid = lax.axis_index("s") * NC + lax.axis_index("c")
    base_a = wid * A_PER
    pltpu.sync_copy(srcpad.at[pl.ds(base_a * DEG, A_PER * DEG)], idx_v)
    pltpu.sync_copy(aproj.at[pl.ds(base_a, A_PER)], ap_v)

    zero = jnp.zeros((16,), jnp.float32)
    for c in range(2 * C):
        acc_v[pl.ds(c * 16, 16)] = zero

    def start(b, rows_v, sem):
        pltpu.async_copy(
            tproj.at[idx_v.at[pl.ds(b * BROWS, BROWS)]], rows_v, sem)

    def wait(rows_v, sem):
        pltpu.make_async_copy(tproj.at[idx_v.at[pl.ds(0, BROWS)]],
                              rows_v, sem).wait()

    def compute(b, rows_v):
        for k in range(AB):
            a = b * AB + k
            valid = (base_a + a) < N_AG

            @pl.when(valid)
            def _():
                arow = [ap_v[a, pl.ds(c * 16, 16)] for c in range(C)]

                def edge_body(j, sq):
                    s, q = sq
                    s2, q2 = [], []
                    for c in range(C):
                        r = rows_v[j, pl.ds(c * 16, 16)]
                        h = r + arow[c]
                        s2.append(s[c] + h)
                        q2.append(q[c] + h * h)
                    return (tuple(s2), tuple(q2))

                init = (
                    tuple(acc_v[pl.ds(c * 16, 16)] for c in range(C)),
                    tuple(acc_v[pl.ds((C + c) * 16, 16)] for c in range(C)),
                )
                fs, fq = lax.fori_loop(k * DEG, (k + 1) * DEG, edge_body, init)
                for c in range(C):
                    acc_v[pl.ds(c * 16, 16)] = fs[c]
                    acc_v[pl.ds((C + c) * 16, 16)] = fq[c]

    start(0, rows0_v, sem0)

    def pair_body(g, carry):
        start(2 * g + 1, rows1_v, sem1)
        wait(rows0_v, sem0)
        compute(2 * g, rows0_v)

        @pl.when(2 * g + 2 < NB)
        def _():
            start(2 * g + 2, rows0_v, sem0)

        wait(rows1_v, sem1)
        compute(2 * g + 1, rows1_v)
        return carry

    lax.fori_loop(0, NB // 2, pair_body, jnp.int32(0))
    pltpu.sync_copy(acc_v, part.at[wid])


@functools.partial(
    pl.kernel,
    out_type=jax.ShapeDtypeStruct((NW, 2 * D), jnp.float32),
    mesh=_SC_MESH,
    compiler_params=pltpu.CompilerParams(needs_layout_passes=False),
    scratch_types=[
        pltpu.VMEM((A_PER * DEG,), jnp.int32),
        pltpu.VMEM((A_PER, D), jnp.float32),
        pltpu.VMEM((BROWS, D), jnp.float32),
        pltpu.VMEM((BROWS, D), jnp.float32),
        pltpu.VMEM((2 * D,), jnp.float32),
        pltpu.SemaphoreType.DMA,
        pltpu.SemaphoreType.DMA,
    ],
)
def _k1(tproj, aproj, srcpad, part, idx_v, ap_v, rows0_v, rows1_v, acc_v,
        sem0, sem1):
    _k1_body(tproj, aproj, srcpad, part, idx_v, ap_v, rows0_v, rows1_v, acc_v,
             sem0, sem1)


def _k2_body(tproj, aproj, srcpad, kbw, fin, s_out, f_out, part2,
             idx_v, ap_v, rows0_v, rows1_v, accb_v, sbuf_v, fbuf_v, kbw_v,
             fin_v, acc2_v, sem0, sem1):
    wid = lax.axis_index("s") * NC + lax.axis_index("c")
    base_a = wid * A_PER
    pltpu.sync_copy(srcpad.at[pl.ds(base_a * DEG, A_PER * DEG)], idx_v)
    pltpu.sync_copy(aproj.at[pl.ds(base_a, A_PER)], ap_v)
    pltpu.sync_copy(kbw, kbw_v)
    pltpu.sync_copy(fin, fin_v)

    zero = jnp.zeros((16,), jnp.float32)
    acc2_v[pl.ds(0, 16)] = zero
    acc2_v[pl.ds(16, 16)] = zero

    k1v = [kbw_v[pl.ds(c * 16, 16)] for c in range(C)]
    b1v = [kbw_v[pl.ds((C + c) * 16, 16)] for c in range(C)]
    w2v = [kbw_v[pl.ds((2 * C + c) * 16, 16)] for c in range(C)]
    lanes = lax.iota(jnp.int32, 16)

    def start(b, rows_v, sem):
        pltpu.async_copy(
            tproj.at[idx_v.at[pl.ds(b * BROWS, BROWS)]], rows_v, sem)

    def wait(rows_v, sem):
        pltpu.make_async_copy(tproj.at[idx_v.at[pl.ds(0, BROWS)]],
                              rows_v, sem).wait()

    def compute(b, rows_v):
        for k in range(AB):
            a = b * AB + k
            valid = (base_a + a) < N_AG

            @pl.when(valid)
            def _():
                cav = [ap_v[a, pl.ds(c * 16, 16)] * k1v[c] + b1v[c]
                       for c in range(C)]

                def edge_body(j, cy):
                    acc0 = zero
                    acc1 = zero
                    for c in range(C):
                        r = rows_v[k * DEG + j, pl.ds(c * 16, 16)]
                        hn = r * k1v[c] + cav[c]
                        lr = jnp.maximum(hn, hn * NEG_SLOPE)
                        t = lr * w2v[c]
                        if c % 2 == 0:
                            acc0 = acc0 + t
                        else:
                            acc1 = acc1 + t
                    accb_v[pl.ds(j * 16, 16)] = acc0 + acc1
                    return cy

                lax.fori_loop(0, DEG, edge_body, jnp.int32(0))

                ss = acc2_v[pl.ds(0, 16)]
                ss2 = acc2_v[pl.ds(16, 16)]
                for g in range(DEG // 16):
                    baseidx = (g * 16) * 16 + lanes * 16
                    sa = plsc.load_gather(accb_v, [baseidx])
                    sb = plsc.load_gather(accb_v, [baseidx + 1])
                    for m in range(2, 16, 2):
                        sa = sa + plsc.load_gather(accb_v, [baseidx + m])
                        sb = sb + plsc.load_gather(accb_v, [baseidx + m + 1])
                    s_grp = sa + sb
                    sbuf_v[pl.ds(a * DEG + g * 16, 16)] = s_grp
                    srcv = idx_v[pl.ds(a * DEG + g * 16, 16)]
                    fbuf_v[pl.ds(a * DEG + g * 16, 16)] = (
                        plsc.load_gather(fin_v, [srcv]))
                    ss = ss + s_grp
                    ss2 = ss2 + s_grp * s_grp
                acc2_v[pl.ds(0, 16)] = ss
                acc2_v[pl.ds(16, 16)] = ss2

    start(0, rows0_v, sem0)

    def pair_body(g, carry):
        start(2 * g + 1, rows1_v, sem1)
        wait(rows0_v, sem0)
        compute(2 * g, rows0_v)

        @pl.when(2 * g + 2 < NB)
        def _():
            start(2 * g + 2, rows0_v, sem0)

        wait(rows1_v, sem1)
        compute(2 * g + 1, rows1_v)
        return carry

    lax.fori_loop(0, NB // 2, pair_body, jnp.int32(0))
    pltpu.sync_copy(sbuf_v, s_out.at[pl.ds(base_a * DEG, A_PER * DEG)])
    pltpu.sync_copy(fbuf_v, f_out.at[pl.ds(base_a * DEG, A_PER * DEG)])
    pltpu.sync_copy(acc2_v, part2.at[wid])


@functools.partial(
    pl.kernel,
    out_type=(
        jax.ShapeDtypeStruct((E_PAD,), jnp.float32),
        jax.ShapeDtypeStruct((E_PAD,), jnp.float32),
        jax.ShapeDtypeStruct((NW, 32), jnp.float32),
    ),
    mesh=_SC_MESH,
    compiler_params=pltpu.CompilerParams(needs_layout_passes=False),
    scratch_types=[
        pltpu.VMEM((A_PER * DEG,), jnp.int32),
        pltpu.VMEM((A_PER, D), jnp.float32),
        pltpu.VMEM((BROWS, D), jnp.float32),
        pltpu.VMEM((BROWS, D), jnp.float32),
        pltpu.VMEM((DEG * 16,), jnp.float32),
        pltpu.VMEM((A_PER * DEG,), jnp.float32),
        pltpu.VMEM((A_PER * DEG,), jnp.float32),
        pltpu.VMEM((3 * D,), jnp.float32),
        pltpu.VMEM((N_TASK,), jnp.float32),
        pltpu.VMEM((32,), jnp.float32),
        pltpu.SemaphoreType.DMA,
        pltpu.SemaphoreType.DMA,
    ],
)
def _k2(tproj, aproj, srcpad, kbw, fin, s_out, f_out, part2,
        idx_v, ap_v, rows0_v, rows1_v, accb_v, sbuf_v, fbuf_v, kbw_v, fin_v,
        acc2_v, sem0, sem1):
    _k2_body(tproj, aproj, srcpad, kbw, fin, s_out, f_out, part2,
             idx_v, ap_v, rows0_v, rows1_v, accb_v, sbuf_v, fbuf_v, kbw_v,
             fin_v, acc2_v, sem0, sem1)


def _fin_body(s_ref, f_ref, scal_ref, o_ref):
    k2 = scal_ref[0]
    b2 = scal_ref[1]
    o_ref[...] = jnp.where(f_ref[...] > 0.5, -jnp.inf,
                           s_ref[...] * k2 + b2)


def _final(s2, f2, scal):
    rows = E // D
    return pl.pallas_call(
        _fin_body,
        grid=(1,),
        in_specs=[
            pl.BlockSpec((rows, D), lambda i: (0, 0)),
            pl.BlockSpec((rows, D), lambda i: (0, 0)),
            pl.BlockSpec(memory_space=pltpu.SMEM),
        ],
        out_specs=pl.BlockSpec((rows, D), lambda i: (0, 0)),
        out_shape=jax.ShapeDtypeStruct((rows, D), jnp.float32),
    )(s2, f2, scal)


def kernel(nf, edge_index, node_type, W1, gamma1, beta1, W2, gamma2, beta2):
    src = edge_index[0].astype(jnp.int32)
    nf_t = nf[:N_TASK]
    nf_a = nf[N_TASK:]
    w1s = W1[:D]
    w1d = W1[D:]

    t_proj, a_proj = _proj(nf_t, nf_a, w1s, w1d)

    src_pad = jnp.pad(src, (0, E_PAD - E))
    a_proj_pad = jnp.pad(a_proj, ((0, N_AG_PAD - N_AG), (0, 0)))

    part = _k1(t_proj, a_proj_pad, src_pad)
    sums = part[:, :D].sum(axis=0)
    sqs = part[:, D:].sum(axis=0)
    mu1 = sums / E
    var1 = sqs / E - mu1 * mu1
    k1 = gamma1 / jnp.sqrt(var1 + EPS)
    b1 = beta1 - mu1 * k1

    kbw = jnp.concatenate([k1, b1, W2[:, 0]])
    fin = (node_type[:N_TASK] == FIN_TASK_TYPE).astype(jnp.float32)

    s_pad, f_pad, part2 = _k2(t_proj, a_proj_pad, src_pad, kbw, fin)
    ss = part2[:, :16].sum()
    ss2 = part2[:, 16:].sum()
    mu2 = ss / E
    var2 = ss2 / E - mu2 * mu2
    k2 = gamma2[0] / jnp.sqrt(var2 + EPS)
    b2 = beta2[0] - mu2 * k2
    scal = jnp.stack([k2, b2])

    s2 = s_pad[:E].reshape(E // D, D)
    f2 = f_pad[:E].reshape(E // D, D)
    out = _final(s2, f2, scal)
    return out.reshape(N_AG, DEG)
